# baseline (speedup 1.0000x reference)
import functools

import jax
import jax.numpy as jnp
from jax import lax
from jax.experimental import pallas as pl
from jax.experimental.pallas import tpu as pltpu
from jax.experimental.pallas import tpu_sc as plsc

D_HALF = 64
D_MODEL = 128


@functools.lru_cache(maxsize=None)
def _make_sc_gather(n_rows: int, chunk: int):
    info = plsc.get_sparse_core_info()
    nc, ns = info.num_cores, info.num_subcores
    nw = nc * ns
    n_per = n_rows // nw
    n_chunks = n_per // chunk
    assert n_per % chunk == 0 and n_rows % nw == 0 and chunk % 8 == 0

    mesh = plsc.VectorSubcoreMesh(core_axis_name="c", subcore_axis_name="s")

    @functools.partial(
        pl.kernel,
        mesh=mesh,
        compiler_params=pltpu.CompilerParams(use_tc_tiling_on_sc=False),
        out_type=jax.ShapeDtypeStruct((n_rows, D_MODEL), jnp.float32),
        scratch_types=[
            pltpu.VMEM((chunk,), jnp.int32),
            pltpu.VMEM((chunk, D_MODEL), jnp.float32),
            pltpu.SemaphoreType.DMA,
        ],
    )
    def gather_k(ids_hbm, table_hbm, out_hbm, idx_v, rows_v, sem):
        wid = lax.axis_index("s") * nc + lax.axis_index("c")
        base = wid * n_per

        def body(i, carry):
            off = base + i * chunk
            pltpu.sync_copy(ids_hbm.at[pl.ds(off, chunk)], idx_v)
            pltpu.async_copy(table_hbm.at[idx_v], rows_v, sem).wait()
            pltpu.sync_copy(rows_v.at[:, pl.ds(0, D_HALF)],
                            out_hbm.at[pl.ds(off, chunk), pl.ds(0, D_HALF)])
            return carry

        lax.fori_loop(0, n_chunks, body, 0)

    return gather_k


def _tc_body(l, id_ref, st_ref, wst_ref, bst_ref, wc_ref, bc_ref, o_ref):
    bb = st_ref.shape[0]
    stats = st_ref[...].reshape(bb * l, 10)
    pre = jnp.dot(stats, wst_ref[...], preferred_element_type=jnp.float32)
    pre = pre + bst_ref[...]
    stat_emb = 0.5 * pre * (1.0 + lax.erf(pre * 0.7071067811865476))
    id_emb = id_ref[...][:, :D_HALF]
    acc = jnp.dot(id_emb, wc_ref[:D_HALF, :],
                  preferred_element_type=jnp.float32)
    acc = acc + jnp.dot(stat_emb, wc_ref[D_HALF:, :],
                        preferred_element_type=jnp.float32)
    o_ref[...] = (acc + bc_ref[...]).reshape(bb, l, D_MODEL)


@functools.lru_cache(maxsize=None)
def _make_tc_dense(b: int, l: int, bb: int):
    assert b % bb == 0
    grid = (b // bb,)
    blk = bb * l
    return pl.pallas_call(
        functools.partial(_tc_body, l),
        grid=grid,
        in_specs=[
            pl.BlockSpec((blk, D_MODEL), lambda i: (i, 0)),
            pl.BlockSpec((bb, l, 10), lambda i: (i, 0, 0)),
            pl.BlockSpec((10, D_HALF), lambda i: (0, 0)),
            pl.BlockSpec((1, D_HALF), lambda i: (0, 0)),
            pl.BlockSpec((D_MODEL, D_MODEL), lambda i: (0, 0)),
            pl.BlockSpec((1, D_MODEL), lambda i: (0, 0)),
        ],
        out_specs=pl.BlockSpec((bb, l, D_MODEL), lambda i: (i, 0, 0)),
        out_shape=jax.ShapeDtypeStruct((b, l, D_MODEL), jnp.float32),
    )


def kernel(card_ids, card_stats, emb_table, W_stat, b_stat, W_comb, b_comb):
    b, l = card_ids.shape
    n_rows = b * l
    flat_ids = card_ids.reshape(n_rows).astype(jnp.int32)

    table128 = jnp.pad(emb_table, ((0, 0), (0, D_MODEL - D_HALF)))
    id_emb = _make_sc_gather(n_rows, 640)(flat_ids, table128)

    return _make_tc_dense(b, l, 64)(
        id_emb,
        card_stats,
        W_stat,
        b_stat.reshape(1, D_HALF),
        W_comb,
        b_comb.reshape(1, D_MODEL),
    )


# speedup vs baseline: 1.0429x; 1.0429x over previous
import functools

import jax
import jax.numpy as jnp
from jax import lax
from jax.experimental import pallas as pl
from jax.experimental.pallas import tpu as pltpu
from jax.experimental.pallas import tpu_sc as plsc

D_HALF = 64
D_MODEL = 128


@functools.lru_cache(maxsize=None)
def _make_sc_gather(n_rows: int, chunk: int):
    info = plsc.get_sparse_core_info()
    nc, ns = info.num_cores, info.num_subcores
    nw = nc * ns
    n_per = n_rows // nw
    n_chunks = n_per // chunk
    assert n_per % chunk == 0 and n_rows % nw == 0 and chunk % 8 == 0

    mesh = plsc.VectorSubcoreMesh(core_axis_name="c", subcore_axis_name="s")

    @functools.partial(
        pl.kernel,
        mesh=mesh,
        compiler_params=pltpu.CompilerParams(use_tc_tiling_on_sc=False),
        out_type=jax.ShapeDtypeStruct((n_rows, D_MODEL), jnp.float32),
        scratch_types=[
            pltpu.VMEM((chunk,), jnp.int32),
            pltpu.VMEM((chunk, D_HALF), jnp.float32),
            pltpu.SemaphoreType.DMA,
        ],
    )
    def gather_k(ids_hbm, table_hbm, out_hbm, idx_v, rows_v, sem):
        wid = lax.axis_index("s") * nc + lax.axis_index("c")
        base = wid * n_per

        def body(i, carry):
            off = base + i * chunk
            pltpu.sync_copy(ids_hbm.at[pl.ds(off, chunk)], idx_v)
            pltpu.async_copy(table_hbm.at[idx_v], rows_v, sem).wait()
            pltpu.sync_copy(rows_v,
                            out_hbm.at[pl.ds(off, chunk), pl.ds(0, D_HALF)])
            return carry

        lax.fori_loop(0, n_chunks, body, 0)

    return gather_k


def _tc_body(l, id_ref, st_ref, wst_ref, bst_ref, wc_ref, bc_ref, o_ref):
    bb = st_ref.shape[0]
    stats = st_ref[...].reshape(bb * l, 10)
    pre = jnp.dot(stats, wst_ref[...], preferred_element_type=jnp.float32)
    pre = pre + bst_ref[...]
    stat_emb = 0.5 * pre * (1.0 + lax.erf(pre * 0.7071067811865476))
    id_emb = id_ref[...][:, :D_HALF]
    acc = jnp.dot(id_emb, wc_ref[:D_HALF, :],
                  preferred_element_type=jnp.float32)
    acc = acc + jnp.dot(stat_emb, wc_ref[D_HALF:, :],
                        preferred_element_type=jnp.float32)
    o_ref[...] = (acc + bc_ref[...]).reshape(bb, l, D_MODEL)


@functools.lru_cache(maxsize=None)
def _make_tc_dense(b: int, l: int, bb: int):
    assert b % bb == 0
    grid = (b // bb,)
    blk = bb * l
    return pl.pallas_call(
        functools.partial(_tc_body, l),
        grid=grid,
        in_specs=[
            pl.BlockSpec((blk, D_MODEL), lambda i: (i, 0)),
            pl.BlockSpec((bb, l, 10), lambda i: (i, 0, 0)),
            pl.BlockSpec((10, D_HALF), lambda i: (0, 0)),
            pl.BlockSpec((1, D_HALF), lambda i: (0, 0)),
            pl.BlockSpec((D_MODEL, D_MODEL), lambda i: (0, 0)),
            pl.BlockSpec((1, D_MODEL), lambda i: (0, 0)),
        ],
        out_specs=pl.BlockSpec((bb, l, D_MODEL), lambda i: (i, 0, 0)),
        out_shape=jax.ShapeDtypeStruct((b, l, D_MODEL), jnp.float32),
    )


def kernel(card_ids, card_stats, emb_table, W_stat, b_stat, W_comb, b_comb):
    b, l = card_ids.shape
    n_rows = b * l
    flat_ids = card_ids.reshape(n_rows).astype(jnp.int32) * 2
    table2 = jnp.pad(emb_table, ((0, 0), (0, D_MODEL - D_HALF)))
    table2 = table2.reshape(2 * table2.shape[0], D_HALF)
    id_emb = _make_sc_gather(n_rows, 640)(flat_ids, table2)

    return _make_tc_dense(b, l, 64)(
        id_emb,
        card_stats,
        W_stat,
        b_stat.reshape(1, D_HALF),
        W_comb,
        b_comb.reshape(1, D_MODEL),
    )
